# Initial kernel scaffold; baseline (speedup 1.0000x reference)
import jax
import jax.numpy as jnp
from jax.experimental import pallas as pl


def kernel(x, edge_index, adj_vals, W0):
    raise NotImplementedError("write your pallas kernel here")


# speedup vs baseline: 4.5579x; 4.5579x over previous
import functools

import jax
import jax.numpy as jnp
from jax import lax
from jax.experimental import pallas as pl
from jax.experimental.pallas import tpu as pltpu
from jax.experimental.pallas import tpu_sc as plsc

N = 10000
E = 320000
D = 128

NC = 2
NS = 16
NW = NC * NS
EPW = E // NW
K = 80
NCHUNK = EPW // K
NROWCHUNK = N // K
LANES = 16


def _sc_aggregate(x, row, col, val):
  mesh = plsc.VectorSubcoreMesh(
      core_axis_name="c", subcore_axis_name="s", num_cores=NC)

  @functools.partial(
      pl.kernel,
      out_type=jax.ShapeDtypeStruct((NC, N, D), jnp.float32),
      mesh=mesh,
      scratch_types=[
          pltpu.VMEM((K,), jnp.int32),
          pltpu.VMEM((K,), jnp.int32),
          pltpu.VMEM((K,), jnp.float32),
          pltpu.VMEM((K, D), jnp.float32),
          pltpu.VMEM_SHARED((N, D), jnp.float32),
          pltpu.SemaphoreType.DMA,
      ],
  )
  def agg_kernel(x_hbm, row_hbm, col_hbm, val_hbm, out_hbm,
                 col_v, row_v, val_v, rows_v, acc, sem):
    cid = lax.axis_index("c")
    sid = lax.axis_index("s")
    wid = cid * NS + sid

    def for_owned_row_chunks(fn):
      def body(t, carry):
        c = sid + t * NS

        @pl.when(c < NROWCHUNK)
        def _():
          fn(pl.multiple_of(c * K, 8))

        return carry

      lax.fori_loop(0, (NROWCHUNK + NS - 1) // NS, body, 0)

    zero16 = jnp.zeros((LANES,), jnp.float32)

    def zero_body(i, carry):
      for j in range(D // LANES):
        rows_v[i, pl.ds(j * LANES, LANES)] = zero16
      return carry

    lax.fori_loop(0, K, zero_body, 0)
    for_owned_row_chunks(lambda rb: pltpu.sync_copy(rows_v, acc.at[pl.ds(rb, K)]))
    plsc.subcore_barrier()

    def chunk_body(i, carry):
      ebase = pl.multiple_of(wid * EPW + i * K, 8)
      pltpu.sync_copy(col_hbm.at[pl.ds(ebase, K)], col_v)
      pltpu.sync_copy(row_hbm.at[pl.ds(ebase, K)], row_v)
      pltpu.sync_copy(val_hbm.at[pl.ds(ebase, K)], val_v)
      pltpu.async_copy(x_hbm.at[col_v], rows_v, sem).wait()

      def scale_body(g, c2):
        vv = val_v[pl.ds(g * LANES, LANES)]
        for l in range(LANES):
          bv = lax.gather(
              vv, jnp.full((LANES, 1), l, jnp.int32),
              lax.GatherDimensionNumbers(offset_dims=(),
                                         collapsed_slice_dims=(0,),
                                         start_index_map=(0,)),
              slice_sizes=(1,),
              mode=lax.GatherScatterMode.PROMISE_IN_BOUNDS)
          e = g * LANES + l
          for j in range(D // LANES):
            sl = (e, pl.ds(j * LANES, LANES))
            rows_v[sl] = rows_v[sl] * bv
        return c2

      lax.fori_loop(0, K // LANES, scale_body, 0)
      pltpu.sync_copy(rows_v, acc.at[row_v], add=True)
      return carry

    lax.fori_loop(0, NCHUNK, chunk_body, 0)
    plsc.subcore_barrier()

    for_owned_row_chunks(
        lambda rb: pltpu.sync_copy(acc.at[pl.ds(rb, K)],
                                   out_hbm.at[cid, pl.ds(rb, K)]))

  return agg_kernel(x, row, col, val)


def _tc_combine_matmul(partials, W0):
  BR = 1000

  def body(p_ref, w_ref, o_ref):
    o_ref[...] = jnp.dot(p_ref[0] + p_ref[1], w_ref[...],
                         preferred_element_type=jnp.float32)

  return pl.pallas_call(
      body,
      grid=(N // BR,),
      in_specs=[
          pl.BlockSpec((NC, BR, D), lambda i: (0, i, 0)),
          pl.BlockSpec((D, D), lambda i: (0, 0)),
      ],
      out_specs=pl.BlockSpec((BR, D), lambda i: (i, 0)),
      out_shape=jax.ShapeDtypeStruct((N, D), jnp.float32),
  )(partials, W0)


@jax.jit
def kernel(x, edge_index, adj_vals, W0):
  row = edge_index[0]
  col = edge_index[1]
  partials = _sc_aggregate(x, row, col, adj_vals)
  return _tc_combine_matmul(partials, W0)


# speedup vs baseline: 10.3431x; 2.2693x over previous
import functools

import jax
import jax.numpy as jnp
from jax import lax
from jax.experimental import pallas as pl
from jax.experimental.pallas import tpu as pltpu
from jax.experimental.pallas import tpu_sc as plsc

N = 10000
E = 320000
D = 128

NC = 2
NS = 16
NW = NC * NS
EPW = E // NW
K = 80
NCHUNK = EPW // K
W = 25
NWIN = NCHUNK // W
NROWCHUNK = N // K
LANES = 16


def _sc_aggregate(x, row3, col3, val3):
  mesh = plsc.VectorSubcoreMesh(
      core_axis_name="c", subcore_axis_name="s", num_cores=NC)

  @functools.partial(
      pl.kernel,
      out_type=jax.ShapeDtypeStruct((NC, N, D), jnp.float32),
      mesh=mesh,
      scratch_types=[
          pltpu.VMEM((W, K), jnp.int32),
          pltpu.VMEM((W, K), jnp.int32),
          pltpu.VMEM((W, K), jnp.float32),
          pltpu.VMEM((K, D), jnp.float32),
          pltpu.VMEM((K, D), jnp.float32),
          pltpu.VMEM_SHARED((N, D), jnp.float32),
          pltpu.SemaphoreType.DMA,
          pltpu.SemaphoreType.DMA,
      ],
  )
  def agg_kernel(x_hbm, row_hbm, col_hbm, val_hbm, out_hbm,
                 col_v, row_v, val_v, rows0, rows1, acc, sem0, sem1):
    cid = lax.axis_index("c")
    sid = lax.axis_index("s")
    wid = cid * NS + sid

    def for_owned_row_chunks(fn):
      def body(t, carry):
        c = sid + t * NS

        @pl.when(c < NROWCHUNK)
        def _():
          fn(pl.multiple_of(c * K, 8))

        return carry

      lax.fori_loop(0, (NROWCHUNK + NS - 1) // NS, body, 0)

    zero16 = jnp.zeros((LANES,), jnp.float32)

    def zero_body(i, carry):
      for j in range(D // LANES):
        rows0[i, pl.ds(j * LANES, LANES)] = zero16
      return carry

    lax.fori_loop(0, K, zero_body, 0)
    for_owned_row_chunks(lambda rb: pltpu.sync_copy(rows0, acc.at[pl.ds(rb, K)]))
    plsc.subcore_barrier()

    def start_gather(i, buf, sem):
      return pltpu.async_copy(x_hbm.at[col_v.at[i]], buf, sem)

    def wait_gather(buf, sem):
      pltpu.make_async_copy(x_hbm.at[pl.ds(0, K)], buf, sem).wait()

    def scale(buf, i):
      def group_body(g, carry):
        vv = val_v[i, pl.ds(g * LANES, LANES)]
        for l in range(LANES):
          bv = lax.gather(
              vv, jnp.full((LANES, 1), l, jnp.int32),
              lax.GatherDimensionNumbers(offset_dims=(),
                                         collapsed_slice_dims=(0,),
                                         start_index_map=(0,)),
              slice_sizes=(1,),
              mode=lax.GatherScatterMode.PROMISE_IN_BOUNDS)
          e = g * LANES + l
          for j in range(D // LANES):
            sl = (e, pl.ds(j * LANES, LANES))
            buf[sl] = buf[sl] * bv
        return carry

      lax.fori_loop(0, K // LANES, group_body, 0)

    def scatter_add(buf, i):
      pltpu.sync_copy(buf, acc.at[row_v.at[i]], add=True)

    def window_body(w, carry):
      pltpu.sync_copy(col_hbm.at[wid, w], col_v)
      pltpu.sync_copy(row_hbm.at[wid, w], row_v)
      pltpu.sync_copy(val_hbm.at[wid, w], val_v)
      start_gather(0, rows0, sem0)
      start_gather(1, rows1, sem1)

      def chunk_pair(g, carry2):
        a = g * 2
        wait_gather(rows0, sem0)
        scale(rows0, a)
        scatter_add(rows0, a)

        @pl.when(a + 2 < W)
        def _():
          start_gather(a + 2, rows0, sem0)

        b = a + 1
        wait_gather(rows1, sem1)
        scale(rows1, b)
        scatter_add(rows1, b)

        @pl.when(b + 2 < W)
        def _():
          start_gather(b + 2, rows1, sem1)

        return carry2

      lax.fori_loop(0, W // 2, chunk_pair, 0)
      wait_gather(rows0, sem0)
      scale(rows0, W - 1)
      scatter_add(rows0, W - 1)
      return carry

    lax.fori_loop(0, NWIN, window_body, 0)

    plsc.subcore_barrier()

    for_owned_row_chunks(
        lambda rb: pltpu.sync_copy(acc.at[pl.ds(rb, K)],
                                   out_hbm.at[cid, pl.ds(rb, K)]))

  return agg_kernel(x, row3, col3, val3)


def _tc_combine_matmul(partials, W0):
  BR = 1000

  def body(p_ref, w_ref, o_ref):
    o_ref[...] = jnp.dot(p_ref[0] + p_ref[1], w_ref[...],
                         preferred_element_type=jnp.float32)

  return pl.pallas_call(
      body,
      grid=(N // BR,),
      in_specs=[
          pl.BlockSpec((NC, BR, D), lambda i: (0, i, 0)),
          pl.BlockSpec((D, D), lambda i: (0, 0)),
      ],
      out_specs=pl.BlockSpec((BR, D), lambda i: (i, 0)),
      out_shape=jax.ShapeDtypeStruct((N, D), jnp.float32),
  )(partials, W0)


@jax.jit
def kernel(x, edge_index, adj_vals, W0):
  row3 = edge_index[0].reshape(NW, NWIN, W, K)
  col3 = edge_index[1].reshape(NW, NWIN, W, K)
  val3 = adj_vals.reshape(NW, NWIN, W, K)
  partials = _sc_aggregate(x, row3, col3, val3)
  return _tc_combine_matmul(partials, W0)


# speedup vs baseline: 11.3991x; 1.1021x over previous
import functools

import jax
import jax.numpy as jnp
from jax import lax
from jax.experimental import pallas as pl
from jax.experimental.pallas import tpu as pltpu
from jax.experimental.pallas import tpu_sc as plsc

N = 10000
E = 320000
D = 128

NC = 2
NS = 16
NW = NC * NS
EPW = E // NW
K = 80
NCHUNK = EPW // K
W = 25
NWIN = NCHUNK // W
NROWCHUNK = N // K
LANES = 16


def _sc_aggregate(x, row3, col3, val3):
  mesh = plsc.VectorSubcoreMesh(
      core_axis_name="c", subcore_axis_name="s", num_cores=NC)

  @functools.partial(
      pl.kernel,
      out_type=jax.ShapeDtypeStruct((NC, N, D), jnp.float32),
      mesh=mesh,
      scratch_types=[
          pltpu.VMEM((W, K), jnp.int32),
          pltpu.VMEM((W, K), jnp.int32),
          pltpu.VMEM((W, K), jnp.float32),
          pltpu.VMEM((K, D), jnp.float32),
          pltpu.VMEM((K, D), jnp.float32),
          pltpu.VMEM((K, D), jnp.float32),
          pltpu.VMEM_SHARED((N, D), jnp.float32),
          pltpu.SemaphoreType.DMA,
          pltpu.SemaphoreType.DMA,
          pltpu.SemaphoreType.DMA,
          pltpu.SemaphoreType.DMA,
          pltpu.SemaphoreType.DMA,
          pltpu.SemaphoreType.DMA,
      ],
  )
  def agg_kernel(x_hbm, row_hbm, col_hbm, val_hbm, out_hbm,
                 col_v, row_v, val_v, rows0, rows1, rows2, acc,
                 sg0, sg1, sg2, ss0, ss1, ss2):
    bufs = (rows0, rows1, rows2)
    gsems = (sg0, sg1, sg2)
    ssems = (ss0, ss1, ss2)
    cid = lax.axis_index("c")
    sid = lax.axis_index("s")
    wid = cid * NS + sid

    def for_owned_row_chunks(fn):
      def body(t, carry):
        c = sid + t * NS

        @pl.when(c < NROWCHUNK)
        def _():
          fn(pl.multiple_of(c * K, 8))

        return carry

      lax.fori_loop(0, (NROWCHUNK + NS - 1) // NS, body, 0)

    zero16 = jnp.zeros((LANES,), jnp.float32)

    def zero_body(i, carry):
      for j in range(D // LANES):
        rows0[i, pl.ds(j * LANES, LANES)] = zero16
      return carry

    lax.fori_loop(0, K, zero_body, 0)
    for_owned_row_chunks(lambda rb: pltpu.sync_copy(rows0, acc.at[pl.ds(rb, K)]))
    plsc.subcore_barrier()

    def start_gather(i, buf, sem):
      return pltpu.async_copy(x_hbm.at[col_v.at[i]], buf, sem)

    def wait_gather(buf, sem):
      pltpu.make_async_copy(x_hbm.at[pl.ds(0, K)], buf, sem).wait()

    def scale(buf, i):
      def group_body(g, carry):
        vv = val_v[i, pl.ds(g * LANES, LANES)]
        for l in range(LANES):
          bv = lax.gather(
              vv, jnp.full((LANES, 1), l, jnp.int32),
              lax.GatherDimensionNumbers(offset_dims=(),
                                         collapsed_slice_dims=(0,),
                                         start_index_map=(0,)),
              slice_sizes=(1,),
              mode=lax.GatherScatterMode.PROMISE_IN_BOUNDS)
          e = g * LANES + l
          for j in range(D // LANES):
            sl = (e, pl.ds(j * LANES, LANES))
            buf[sl] = buf[sl] * bv
        return carry

      lax.fori_loop(0, K // LANES, group_body, 0)

    def start_scatter(i, buf, sem):
      pltpu.async_copy(buf, acc.at[row_v.at[i]], sem, add=True)

    def wait_scatter(buf, sem):
      pltpu.make_async_copy(buf, acc.at[row_v.at[0]], sem).wait()

    def window_body(w, carry):
      pltpu.sync_copy(col_hbm.at[wid, w], col_v)
      pltpu.sync_copy(row_hbm.at[wid, w], row_v)
      pltpu.sync_copy(val_hbm.at[wid, w], val_v)
      start_gather(0, bufs[0], gsems[0])
      start_gather(1, bufs[1], gsems[1])

      def chunk_triple(g, carry2):
        for k in range(3):
          i = g * 3 + k
          wait_gather(bufs[k], gsems[k])
          scale(bufs[k], i)
          start_scatter(i, bufs[k], ssems[k])
          nxt = (k + 2) % 3
          if k == 0:
            @pl.when(g > 0)
            def _():
              wait_scatter(bufs[nxt], ssems[nxt])

            start_gather(i + 2, bufs[nxt], gsems[nxt])
          elif k == 1:
            wait_scatter(bufs[nxt], ssems[nxt])
            start_gather(i + 2, bufs[nxt], gsems[nxt])
          else:
            @pl.when(i + 2 < W)
            def _():
              wait_scatter(bufs[nxt], ssems[nxt])
              start_gather(i + 2, bufs[nxt], gsems[nxt])

        return carry2

      lax.fori_loop(0, (W - 1) // 3, chunk_triple, 0)
      wait_gather(bufs[0], gsems[0])
      scale(bufs[0], W - 1)
      start_scatter(W - 1, bufs[0], ssems[0])
      wait_scatter(bufs[1], ssems[1])
      wait_scatter(bufs[2], ssems[2])
      wait_scatter(bufs[0], ssems[0])
      return carry

    lax.fori_loop(0, NWIN, window_body, 0)

    plsc.subcore_barrier()

    for_owned_row_chunks(
        lambda rb: pltpu.sync_copy(acc.at[pl.ds(rb, K)],
                                   out_hbm.at[cid, pl.ds(rb, K)]))

  return agg_kernel(x, row3, col3, val3)


def _tc_combine_matmul(partials, W0):
  BR = 1000

  def body(p_ref, w_ref, o_ref):
    o_ref[...] = jnp.dot(p_ref[0] + p_ref[1], w_ref[...],
                         preferred_element_type=jnp.float32)

  return pl.pallas_call(
      body,
      grid=(N // BR,),
      in_specs=[
          pl.BlockSpec((NC, BR, D), lambda i: (0, i, 0)),
          pl.BlockSpec((D, D), lambda i: (0, 0)),
      ],
      out_specs=pl.BlockSpec((BR, D), lambda i: (i, 0)),
      out_shape=jax.ShapeDtypeStruct((N, D), jnp.float32),
  )(partials, W0)


@jax.jit
def kernel(x, edge_index, adj_vals, W0):
  row3 = edge_index[0].reshape(NW, NWIN, W, K)
  col3 = edge_index[1].reshape(NW, NWIN, W, K)
  val3 = adj_vals.reshape(NW, NWIN, W, K)
  partials = _sc_aggregate(x, row3, col3, val3)
  return _tc_combine_matmul(partials, W0)


# speedup vs baseline: 11.6785x; 1.0245x over previous
import functools

import jax
import jax.numpy as jnp
from jax import lax
from jax.experimental import pallas as pl
from jax.experimental.pallas import tpu as pltpu
from jax.experimental.pallas import tpu_sc as plsc

N = 10000
E = 320000
D = 128

NC = 2
NS = 16
NW = NC * NS
EPW = E // NW
K = 80
NCHUNK = EPW // K
W = 25
NWIN = NCHUNK // W
NROWCHUNK = N // K
LANES = 16


def _sc_aggregate(x, row3, col3, val3):
  mesh = plsc.VectorSubcoreMesh(
      core_axis_name="c", subcore_axis_name="s", num_cores=NC)

  @functools.partial(
      pl.kernel,
      out_type=jax.ShapeDtypeStruct((NC, N, D), jnp.float32),
      mesh=mesh,
      scratch_types=[
          pltpu.VMEM((W * K,), jnp.int32),
          pltpu.VMEM((W, K), jnp.int32),
          pltpu.VMEM((W * K,), jnp.float32),
          pltpu.VMEM((K, D), jnp.float32),
          pltpu.VMEM((K, D), jnp.float32),
          pltpu.VMEM((K, D), jnp.float32),
          pltpu.VMEM((K, D), jnp.float32),
          pltpu.VMEM_SHARED((N, D), jnp.float32),
          pltpu.SemaphoreType.DMA,
          pltpu.SemaphoreType.DMA,
          pltpu.SemaphoreType.DMA,
          pltpu.SemaphoreType.DMA,
          pltpu.SemaphoreType.DMA,
          pltpu.SemaphoreType.DMA,
          pltpu.SemaphoreType.DMA,
          pltpu.SemaphoreType.DMA,
      ],
  )
  def agg_kernel(x_hbm, row_hbm, col_hbm, val_hbm, out_hbm,
                 col_v, row_v, val_v, rows0, rows1, rows2, rows3, acc,
                 sg0, sg1, sg2, sg3, ss0, ss1, ss2, ss3):
    bufs = (rows0, rows1, rows2, rows3)
    gsems = (sg0, sg1, sg2, sg3)
    ssems = (ss0, ss1, ss2, ss3)
    cid = lax.axis_index("c")
    sid = lax.axis_index("s")
    wid = cid * NS + sid

    def for_owned_row_chunks(fn):
      def body(t, carry):
        c = sid + t * NS

        @pl.when(c < NROWCHUNK)
        def _():
          fn(pl.multiple_of(c * K, 8))

        return carry

      lax.fori_loop(0, (NROWCHUNK + NS - 1) // NS, body, 0)

    zero16 = jnp.zeros((LANES,), jnp.float32)

    def zero_body(i, carry):
      for j in range(D // LANES):
        rows0[i, pl.ds(j * LANES, LANES)] = zero16
      return carry

    lax.fori_loop(0, K, zero_body, 0)
    for_owned_row_chunks(lambda rb: pltpu.sync_copy(rows0, acc.at[pl.ds(rb, K)]))
    plsc.subcore_barrier()

    def start_gather(i, buf, sem):
      idx = col_v.at[pl.ds(pl.multiple_of(i * K, 8), K)]
      return pltpu.async_copy(x_hbm.at[idx], buf, sem)

    def wait_gather(buf, sem):
      pltpu.make_async_copy(x_hbm.at[pl.ds(0, K)], buf, sem).wait()

    def scale(buf, i):
      def group_body(g, carry):
        vv = val_v[pl.ds(i * K + g * LANES, LANES)]
        for l in range(LANES):
          bv = lax.gather(
              vv, jnp.full((LANES, 1), l, jnp.int32),
              lax.GatherDimensionNumbers(offset_dims=(),
                                         collapsed_slice_dims=(0,),
                                         start_index_map=(0,)),
              slice_sizes=(1,),
              mode=lax.GatherScatterMode.PROMISE_IN_BOUNDS)
          e = g * LANES + l
          for j in range(D // LANES):
            sl = (e, pl.ds(j * LANES, LANES))
            buf[sl] = buf[sl] * bv
        return carry

      lax.fori_loop(0, K // LANES, group_body, 0)

    def start_scatter(i, buf, sem):
      pltpu.async_copy(buf, acc.at[row_v.at[i]], sem, add=True)

    def wait_scatter(buf, sem):
      pltpu.make_async_copy(buf, acc.at[row_v.at[0]], sem).wait()

    def window_body(w, carry):
      pltpu.sync_copy(col_hbm.at[wid, w], col_v)
      pltpu.sync_copy(row_hbm.at[wid, w], row_v)
      pltpu.sync_copy(val_hbm.at[wid, w], val_v)
      start_gather(0, bufs[0], gsems[0])
      start_gather(1, bufs[1], gsems[1])
      start_gather(2, bufs[2], gsems[2])

      def chunk_quad(g, carry2):
        for k in range(4):
          i = g * 4 + k
          nxt = (k + 3) % 4
          wait_gather(bufs[k], gsems[k])
          scale(bufs[k], i)
          start_scatter(i, bufs[k], ssems[k])
          if k == 0:
            @pl.when(g > 0)
            def _():
              wait_scatter(bufs[nxt], ssems[nxt])

            start_gather(i + 3, bufs[nxt], gsems[nxt])
          elif k == 1:
            wait_scatter(bufs[nxt], ssems[nxt])
            start_gather(i + 3, bufs[nxt], gsems[nxt])
          else:
            @pl.when(i + 3 < W)
            def _():
              wait_scatter(bufs[nxt], ssems[nxt])
              start_gather(i + 3, bufs[nxt], gsems[nxt])

        return carry2

      lax.fori_loop(0, (W - 1) // 4, chunk_quad, 0)
      wait_gather(bufs[0], gsems[0])
      scale(bufs[0], W - 1)
      start_scatter(W - 1, bufs[0], ssems[0])
      wait_scatter(bufs[1], ssems[1])
      wait_scatter(bufs[2], ssems[2])
      wait_scatter(bufs[3], ssems[3])
      wait_scatter(bufs[0], ssems[0])
      return carry

    lax.fori_loop(0, NWIN, window_body, 0)

    plsc.subcore_barrier()

    for_owned_row_chunks(
        lambda rb: pltpu.sync_copy(acc.at[pl.ds(rb, K)],
                                   out_hbm.at[cid, pl.ds(rb, K)]))

  return agg_kernel(x, row3, col3, val3)


def _tc_combine_matmul(partials, W0):
  BR = 1000

  def body(p_ref, w_ref, o_ref):
    o_ref[...] = jnp.dot(p_ref[0] + p_ref[1], w_ref[...],
                         preferred_element_type=jnp.float32)

  return pl.pallas_call(
      body,
      grid=(N // BR,),
      in_specs=[
          pl.BlockSpec((NC, BR, D), lambda i: (0, i, 0)),
          pl.BlockSpec((D, D), lambda i: (0, 0)),
      ],
      out_specs=pl.BlockSpec((BR, D), lambda i: (i, 0)),
      out_shape=jax.ShapeDtypeStruct((N, D), jnp.float32),
  )(partials, W0)


@jax.jit
def kernel(x, edge_index, adj_vals, W0):
  row3 = edge_index[0].reshape(NW, NWIN, W, K)
  col3 = edge_index[1].reshape(NW, NWIN, W * K)
  val3 = adj_vals.reshape(NW, NWIN, W * K)
  partials = _sc_aggregate(x, row3, col3, val3)
  return _tc_combine_matmul(partials, W0)


# speedup vs baseline: 12.0710x; 1.0336x over previous
import functools

import jax
import jax.numpy as jnp
from jax import lax
from jax.experimental import pallas as pl
from jax.experimental.pallas import tpu as pltpu
from jax.experimental.pallas import tpu_sc as plsc

N = 10000
E = 320000
D = 128

NC = 2
NS = 16
NW = NC * NS
EPW = E // NW
K = 80
NCHUNK = EPW // K
W = 25
NWIN = NCHUNK // W
NROWCHUNK = N // K
LANES = 16


def _sc_aggregate(x, row3, col3, val3):
  mesh = plsc.VectorSubcoreMesh(
      core_axis_name="c", subcore_axis_name="s", num_cores=NC)

  @functools.partial(
      pl.kernel,
      out_type=jax.ShapeDtypeStruct((NC, N, D), jnp.float32),
      mesh=mesh,
      scratch_types=[
          pltpu.VMEM((W * K,), jnp.int32),
          pltpu.VMEM((W, K), jnp.int32),
          pltpu.VMEM((W * K,), jnp.float32),
          pltpu.VMEM((K, D), jnp.float32),
          pltpu.VMEM((K, D), jnp.float32),
          pltpu.VMEM((K, D), jnp.float32),
          pltpu.VMEM((K, D), jnp.float32),
          pltpu.VMEM_SHARED((N, D), jnp.float32),
          pltpu.SemaphoreType.DMA,
          pltpu.SemaphoreType.DMA,
          pltpu.SemaphoreType.DMA,
          pltpu.SemaphoreType.DMA,
          pltpu.SemaphoreType.DMA,
          pltpu.SemaphoreType.DMA,
          pltpu.SemaphoreType.DMA,
          pltpu.SemaphoreType.DMA,
      ],
  )
  def agg_kernel(x_hbm, row_hbm, col_hbm, val_hbm, out_hbm,
                 col_v, row_v, val_v, rows0, rows1, rows2, rows3, acc,
                 sg0, sg1, sg2, sg3, ss0, ss1, ss2, ss3):
    bufs = (rows0, rows1, rows2, rows3)
    gsems = (sg0, sg1, sg2, sg3)
    ssems = (ss0, ss1, ss2, ss3)
    cid = lax.axis_index("c")
    sid = lax.axis_index("s")
    wid = cid * NS + sid

    def for_owned_row_chunks(fn):
      def body(t, carry):
        c = sid + t * NS

        @pl.when(c < NROWCHUNK)
        def _():
          fn(pl.multiple_of(c * K, 8))

        return carry

      lax.fori_loop(0, (NROWCHUNK + NS - 1) // NS, body, 0)

    def start_gather(i, buf, sem):
      idx = col_v.at[pl.ds(pl.multiple_of(i * K, 8), K)]
      return pltpu.async_copy(x_hbm.at[idx], buf, sem)

    def wait_gather(buf, sem):
      pltpu.make_async_copy(x_hbm.at[pl.ds(0, K)], buf, sem).wait()

    def scale(buf, i):
      def group_body(g, carry):
        vv = val_v[pl.ds(i * K + g * LANES, LANES)]
        for l in range(LANES):
          bv = lax.gather(
              vv, jnp.full((LANES, 1), l, jnp.int32),
              lax.GatherDimensionNumbers(offset_dims=(),
                                         collapsed_slice_dims=(0,),
                                         start_index_map=(0,)),
              slice_sizes=(1,),
              mode=lax.GatherScatterMode.PROMISE_IN_BOUNDS)
          e = g * LANES + l
          for j in range(D // LANES):
            sl = (e, pl.ds(j * LANES, LANES))
            buf[sl] = buf[sl] * bv
        return carry

      lax.fori_loop(0, K // LANES, group_body, 0)

    def start_scatter(i, buf, sem):
      pltpu.async_copy(buf, acc.at[row_v.at[i]], sem, add=True)

    def wait_scatter(buf, sem):
      pltpu.make_async_copy(buf, acc.at[row_v.at[0]], sem).wait()

    def stage_window_and_prime(w):
      ebase = pl.multiple_of(wid * EPW + w * (W * K), 8)
      pltpu.sync_copy(col_hbm.at[pl.ds(ebase, W * K)], col_v)
      pltpu.sync_copy(row_hbm.at[wid, w], row_v)
      pltpu.sync_copy(val_hbm.at[pl.ds(ebase, W * K)], val_v)
      start_gather(0, bufs[0], gsems[0])
      start_gather(1, bufs[1], gsems[1])
      start_gather(2, bufs[2], gsems[2])

    stage_window_and_prime(0)
    zero16 = jnp.zeros((LANES,), jnp.float32)

    def zero_body(i, carry):
      for j in range(D // LANES):
        rows3[i, pl.ds(j * LANES, LANES)] = zero16
      return carry

    lax.fori_loop(0, K, zero_body, 0)
    for_owned_row_chunks(lambda rb: pltpu.sync_copy(rows3, acc.at[pl.ds(rb, K)]))
    plsc.subcore_barrier()

    def window_body(w, carry):
      def chunk_quad(g, carry2):
        for k in range(4):
          i = g * 4 + k
          nxt = (k + 3) % 4
          wait_gather(bufs[k], gsems[k])
          scale(bufs[k], i)
          start_scatter(i, bufs[k], ssems[k])
          if k == 0:
            @pl.when(g > 0)
            def _():
              wait_scatter(bufs[nxt], ssems[nxt])

            start_gather(i + 3, bufs[nxt], gsems[nxt])
          elif k == 1:
            wait_scatter(bufs[nxt], ssems[nxt])
            start_gather(i + 3, bufs[nxt], gsems[nxt])
          else:
            @pl.when(i + 3 < W)
            def _():
              wait_scatter(bufs[nxt], ssems[nxt])
              start_gather(i + 3, bufs[nxt], gsems[nxt])

        return carry2

      lax.fori_loop(0, (W - 1) // 4, chunk_quad, 0)
      wait_gather(bufs[0], gsems[0])
      scale(bufs[0], W - 1)
      start_scatter(W - 1, bufs[0], ssems[0])
      wait_scatter(bufs[1], ssems[1])
      wait_scatter(bufs[2], ssems[2])
      wait_scatter(bufs[3], ssems[3])
      wait_scatter(bufs[0], ssems[0])

      @pl.when(w + 1 < NWIN)
      def _():
        stage_window_and_prime(w + 1)

      return carry

    lax.fori_loop(0, NWIN, window_body, 0)

    plsc.subcore_barrier()

    for_owned_row_chunks(
        lambda rb: pltpu.sync_copy(acc.at[pl.ds(rb, K)],
                                   out_hbm.at[cid, pl.ds(rb, K)]))

  return agg_kernel(x, row3, col3, val3)


def _tc_combine_matmul(partials, W0):
  BR = 1000

  def body(p_ref, w_ref, o_ref):
    o_ref[...] = jnp.dot(p_ref[0] + p_ref[1], w_ref[...],
                         preferred_element_type=jnp.float32)

  return pl.pallas_call(
      body,
      grid=(N // BR,),
      in_specs=[
          pl.BlockSpec((NC, BR, D), lambda i: (0, i, 0)),
          pl.BlockSpec((D, D), lambda i: (0, 0)),
      ],
      out_specs=pl.BlockSpec((BR, D), lambda i: (i, 0)),
      out_shape=jax.ShapeDtypeStruct((N, D), jnp.float32),
  )(partials, W0)


@jax.jit
def kernel(x, edge_index, adj_vals, W0):
  row3 = edge_index[0].reshape(NW, NWIN, W, K)
  partials = _sc_aggregate(x, row3, edge_index[1], adj_vals)
  return _tc_combine_matmul(partials, W0)


# speedup vs baseline: 12.2381x; 1.0138x over previous
import functools

import jax
import jax.numpy as jnp
from jax import lax
from jax.experimental import pallas as pl
from jax.experimental.pallas import tpu as pltpu
from jax.experimental.pallas import tpu_sc as plsc

N = 10000
E = 320000
D = 128

NC = 2
NS = 16
NW = NC * NS
EPW = E // NW
K = 80
NCHUNK = EPW // K
W = 25
NWIN = NCHUNK // W
NROWCHUNK = N // K
LANES = 16


def _sc_aggregate(x, row3, col3, val3):
  mesh = plsc.VectorSubcoreMesh(
      core_axis_name="c", subcore_axis_name="s", num_cores=NC)

  @functools.partial(
      pl.kernel,
      out_type=jax.ShapeDtypeStruct((NC, N, D), jnp.float32),
      mesh=mesh,
      scratch_types=[
          pltpu.VMEM((W * K,), jnp.int32),
          pltpu.VMEM((W * K,), jnp.int32),
          pltpu.VMEM((W * K,), jnp.float32),
          pltpu.VMEM((K, D), jnp.float32),
          pltpu.VMEM((K, D), jnp.float32),
          pltpu.VMEM((K, D), jnp.float32),
          pltpu.VMEM((K, D), jnp.float32),
          pltpu.VMEM_SHARED((N, D), jnp.float32),
          pltpu.SemaphoreType.DMA,
          pltpu.SemaphoreType.DMA,
          pltpu.SemaphoreType.DMA,
          pltpu.SemaphoreType.DMA,
          pltpu.SemaphoreType.DMA,
          pltpu.SemaphoreType.DMA,
          pltpu.SemaphoreType.DMA,
          pltpu.SemaphoreType.DMA,
      ],
  )
  def agg_kernel(x_hbm, row_hbm, col_hbm, val_hbm, out_hbm,
                 col_v, row_v, val_v, rows0, rows1, rows2, rows3, acc,
                 sg0, sg1, sg2, sg3, ss0, ss1, ss2, ss3):
    bufs = (rows0, rows1, rows2, rows3)
    gsems = (sg0, sg1, sg2, sg3)
    ssems = (ss0, ss1, ss2, ss3)
    cid = lax.axis_index("c")
    sid = lax.axis_index("s")
    wid = cid * NS + sid

    def for_owned_row_chunks(fn):
      def body(t, carry):
        c = sid + t * NS

        @pl.when(c < NROWCHUNK)
        def _():
          fn(pl.multiple_of(c * K, 8))

        return carry

      lax.fori_loop(0, (NROWCHUNK + NS - 1) // NS, body, 0)

    def start_gather(i, buf, sem):
      idx = col_v.at[pl.ds(pl.multiple_of(i * K, 8), K)]
      return pltpu.async_copy(x_hbm.at[idx], buf, sem)

    def wait_gather(buf, sem):
      pltpu.make_async_copy(x_hbm.at[pl.ds(0, K)], buf, sem).wait()

    def scale(buf, i):
      def group_body(g, carry):
        vv = val_v[pl.ds(i * K + g * LANES, LANES)]
        for l in range(LANES):
          bv = lax.gather(
              vv, jnp.full((LANES, 1), l, jnp.int32),
              lax.GatherDimensionNumbers(offset_dims=(),
                                         collapsed_slice_dims=(0,),
                                         start_index_map=(0,)),
              slice_sizes=(1,),
              mode=lax.GatherScatterMode.PROMISE_IN_BOUNDS)
          e = g * LANES + l
          for j in range(D // LANES):
            sl = (e, pl.ds(j * LANES, LANES))
            buf[sl] = buf[sl] * bv
        return carry

      lax.fori_loop(0, K // LANES, group_body, 0)

    def start_scatter(i, buf, sem):
      idx = row_v.at[pl.ds(pl.multiple_of(i * K, 8), K)]
      pltpu.async_copy(buf, acc.at[idx], sem, add=True)

    def wait_scatter(buf, sem):
      pltpu.make_async_copy(buf, acc.at[row_v.at[pl.ds(0, K)]], sem).wait()

    def stage_window_and_prime(w):
      ebase = pl.multiple_of(wid * EPW + w * (W * K), 8)
      pltpu.sync_copy(col_hbm.at[pl.ds(ebase, W * K)], col_v)
      pltpu.sync_copy(row_hbm.at[pl.ds(ebase, W * K)], row_v)
      pltpu.sync_copy(val_hbm.at[pl.ds(ebase, W * K)], val_v)
      start_gather(0, bufs[0], gsems[0])
      start_gather(1, bufs[1], gsems[1])
      start_gather(2, bufs[2], gsems[2])

    stage_window_and_prime(0)
    zero16 = jnp.zeros((LANES,), jnp.float32)

    def zero_body(i, carry):
      for j in range(D // LANES):
        rows3[i, pl.ds(j * LANES, LANES)] = zero16
      return carry

    lax.fori_loop(0, K, zero_body, 0)
    for_owned_row_chunks(lambda rb: pltpu.sync_copy(rows3, acc.at[pl.ds(rb, K)]))
    plsc.subcore_barrier()

    def window_body(w, carry):
      def chunk_quad(g, carry2):
        for k in range(4):
          i = g * 4 + k
          nxt = (k + 3) % 4
          wait_gather(bufs[k], gsems[k])
          scale(bufs[k], i)
          start_scatter(i, bufs[k], ssems[k])
          if k == 0:
            @pl.when(g > 0)
            def _():
              wait_scatter(bufs[nxt], ssems[nxt])

            start_gather(i + 3, bufs[nxt], gsems[nxt])
          elif k == 1:
            wait_scatter(bufs[nxt], ssems[nxt])
            start_gather(i + 3, bufs[nxt], gsems[nxt])
          else:
            @pl.when(i + 3 < W)
            def _():
              wait_scatter(bufs[nxt], ssems[nxt])
              start_gather(i + 3, bufs[nxt], gsems[nxt])

        return carry2

      lax.fori_loop(0, (W - 1) // 4, chunk_quad, 0)
      wait_gather(bufs[0], gsems[0])
      scale(bufs[0], W - 1)
      start_scatter(W - 1, bufs[0], ssems[0])
      wait_scatter(bufs[1], ssems[1])
      wait_scatter(bufs[2], ssems[2])
      wait_scatter(bufs[3], ssems[3])
      wait_scatter(bufs[0], ssems[0])

      @pl.when(w + 1 < NWIN)
      def _():
        stage_window_and_prime(w + 1)

      return carry

    lax.fori_loop(0, NWIN, window_body, 0)

    plsc.subcore_barrier()

    for_owned_row_chunks(
        lambda rb: pltpu.sync_copy(acc.at[pl.ds(rb, K)],
                                   out_hbm.at[cid, pl.ds(rb, K)]))

  return agg_kernel(x, row3, col3, val3)


def _tc_combine_matmul(partials, W0):
  BR = 1000

  def body(p_ref, w_ref, o_ref):
    o_ref[...] = jnp.dot(p_ref[0] + p_ref[1], w_ref[...],
                         preferred_element_type=jnp.float32)

  return pl.pallas_call(
      body,
      grid=(N // BR,),
      in_specs=[
          pl.BlockSpec((NC, BR, D), lambda i: (0, i, 0)),
          pl.BlockSpec((D, D), lambda i: (0, 0)),
      ],
      out_specs=pl.BlockSpec((BR, D), lambda i: (i, 0)),
      out_shape=jax.ShapeDtypeStruct((N, D), jnp.float32),
  )(partials, W0)


@jax.jit
def kernel(x, edge_index, adj_vals, W0):
  partials = _sc_aggregate(x, edge_index[0], edge_index[1], adj_vals)
  return _tc_combine_matmul(partials, W0)


# speedup vs baseline: 13.0328x; 1.0649x over previous
import functools

import jax
import jax.numpy as jnp
from jax import lax
from jax.experimental import pallas as pl
from jax.experimental.pallas import tpu as pltpu
from jax.experimental.pallas import tpu_sc as plsc

N = 10000
E = 320000
D = 128

NC = 2
NS = 16
NW = NC * NS
EPW = E // NW
K = 80
NCHUNK = EPW // K
W = 25
NWIN = NCHUNK // W
NROWCHUNK = N // K
LANES = 16


def _sc_aggregate(x, edge_flat, val3):
  mesh = plsc.VectorSubcoreMesh(
      core_axis_name="c", subcore_axis_name="s", num_cores=NC)

  @functools.partial(
      pl.kernel,
      out_type=jax.ShapeDtypeStruct((NC, N, D), jnp.float32),
      mesh=mesh,
      scratch_types=[
          pltpu.VMEM((W * K,), jnp.int32),
          pltpu.VMEM((W * K,), jnp.int32),
          pltpu.VMEM((W * K,), jnp.float32),
          pltpu.VMEM((K, D), jnp.float32),
          pltpu.VMEM((K, D), jnp.float32),
          pltpu.VMEM((K, D), jnp.float32),
          pltpu.VMEM((K, D), jnp.float32),
          pltpu.VMEM_SHARED((N, D), jnp.float32),
          pltpu.SemaphoreType.DMA,
          pltpu.SemaphoreType.DMA,
          pltpu.SemaphoreType.DMA,
          pltpu.SemaphoreType.DMA,
          pltpu.SemaphoreType.DMA,
          pltpu.SemaphoreType.DMA,
          pltpu.SemaphoreType.DMA,
          pltpu.SemaphoreType.DMA,
      ],
  )
  def agg_kernel(x_hbm, edge_hbm, val_hbm, out_hbm,
                 col_v, row_v, val_v, rows0, rows1, rows2, rows3, acc,
                 sg0, sg1, sg2, sg3, ss0, ss1, ss2, ss3):
    bufs = (rows0, rows1, rows2, rows3)
    gsems = (sg0, sg1, sg2, sg3)
    ssems = (ss0, ss1, ss2, ss3)
    cid = lax.axis_index("c")
    sid = lax.axis_index("s")
    wid = cid * NS + sid

    def for_owned_row_chunks(fn):
      def body(t, carry):
        c = sid + t * NS

        @pl.when(c < NROWCHUNK)
        def _():
          fn(pl.multiple_of(c * K, 8))

        return carry

      lax.fori_loop(0, (NROWCHUNK + NS - 1) // NS, body, 0)

    def start_gather(i, buf, sem):
      idx = col_v.at[pl.ds(pl.multiple_of(i * K, 8), K)]
      return pltpu.async_copy(x_hbm.at[idx], buf, sem)

    def wait_gather(buf, sem):
      pltpu.make_async_copy(x_hbm.at[pl.ds(0, K)], buf, sem).wait()

    def scale(buf, i):
      def group_body(g, carry):
        vv = val_v[pl.ds(i * K + g * LANES, LANES)]
        for l in range(LANES):
          bv = lax.gather(
              vv, jnp.full((LANES, 1), l, jnp.int32),
              lax.GatherDimensionNumbers(offset_dims=(),
                                         collapsed_slice_dims=(0,),
                                         start_index_map=(0,)),
              slice_sizes=(1,),
              mode=lax.GatherScatterMode.PROMISE_IN_BOUNDS)
          e = g * LANES + l
          for j in range(D // LANES):
            sl = (e, pl.ds(j * LANES, LANES))
            buf[sl] = buf[sl] * bv
        return carry

      lax.fori_loop(0, K // LANES, group_body, 0)

    def start_scatter(i, buf, sem):
      idx = row_v.at[pl.ds(pl.multiple_of(i * K, 8), K)]
      pltpu.async_copy(buf, acc.at[idx], sem, add=True)

    def wait_scatter(buf, sem):
      pltpu.make_async_copy(buf, acc.at[row_v.at[pl.ds(0, K)]], sem).wait()

    def stage_window_and_prime(w):
      ebase = pl.multiple_of(wid * EPW + w * (W * K), 8)
      pltpu.sync_copy(edge_hbm.at[pl.ds(E + ebase, W * K)], col_v)
      pltpu.sync_copy(edge_hbm.at[pl.ds(ebase, W * K)], row_v)
      pltpu.sync_copy(val_hbm.at[pl.ds(ebase, W * K)], val_v)
      start_gather(0, bufs[0], gsems[0])
      start_gather(1, bufs[1], gsems[1])
      start_gather(2, bufs[2], gsems[2])

    stage_window_and_prime(0)
    zero16 = jnp.zeros((LANES,), jnp.float32)

    def zero_body(i, carry):
      for j in range(D // LANES):
        rows3[i, pl.ds(j * LANES, LANES)] = zero16
      return carry

    lax.fori_loop(0, K, zero_body, 0)
    for_owned_row_chunks(lambda rb: pltpu.sync_copy(rows3, acc.at[pl.ds(rb, K)]))
    plsc.subcore_barrier()

    def window_body(w, carry):
      def chunk_quad(g, carry2):
        for k in range(4):
          i = g * 4 + k
          nxt = (k + 3) % 4
          wait_gather(bufs[k], gsems[k])
          scale(bufs[k], i)
          start_scatter(i, bufs[k], ssems[k])
          if k == 0:
            @pl.when(g > 0)
            def _():
              wait_scatter(bufs[nxt], ssems[nxt])

            start_gather(i + 3, bufs[nxt], gsems[nxt])
          elif k == 1:
            wait_scatter(bufs[nxt], ssems[nxt])
            start_gather(i + 3, bufs[nxt], gsems[nxt])
          else:
            @pl.when(i + 3 < W)
            def _():
              wait_scatter(bufs[nxt], ssems[nxt])
              start_gather(i + 3, bufs[nxt], gsems[nxt])

        return carry2

      lax.fori_loop(0, (W - 1) // 4, chunk_quad, 0)
      wait_gather(bufs[0], gsems[0])
      scale(bufs[0], W - 1)
      start_scatter(W - 1, bufs[0], ssems[0])
      wait_scatter(bufs[1], ssems[1])
      wait_scatter(bufs[2], ssems[2])
      wait_scatter(bufs[3], ssems[3])
      wait_scatter(bufs[0], ssems[0])

      @pl.when(w + 1 < NWIN)
      def _():
        stage_window_and_prime(w + 1)

      return carry

    lax.fori_loop(0, NWIN, window_body, 0)

    plsc.subcore_barrier()

    for_owned_row_chunks(
        lambda rb: pltpu.sync_copy(acc.at[pl.ds(rb, K)],
                                   out_hbm.at[cid, pl.ds(rb, K)]))

  return agg_kernel(x, edge_flat, val3)


def _tc_combine_matmul(partials, W0):
  BR = 1000

  def body(p_ref, w_ref, o_ref):
    o_ref[...] = jnp.dot(p_ref[0] + p_ref[1], w_ref[...],
                         preferred_element_type=jnp.float32)

  return pl.pallas_call(
      body,
      grid=(N // BR,),
      in_specs=[
          pl.BlockSpec((NC, BR, D), lambda i: (0, i, 0)),
          pl.BlockSpec((D, D), lambda i: (0, 0)),
      ],
      out_specs=pl.BlockSpec((BR, D), lambda i: (i, 0)),
      out_shape=jax.ShapeDtypeStruct((N, D), jnp.float32),
  )(partials, W0)


@jax.jit
def kernel(x, edge_index, adj_vals, W0):
  partials = _sc_aggregate(x, edge_index.reshape(2 * E), adj_vals)
  return _tc_combine_matmul(partials, W0)
